# baseline (device time: 41515 ns/iter reference)
import jax
import jax.numpy as jnp
from jax import lax
from jax.experimental import pallas as pl
from jax.experimental.pallas import tpu as pltpu

T = 512
D = 512
F = 1024
E_LOC = 2


def kernel(x, assign, W1, W2):
    assign2 = assign.reshape(T, 1)

    def body(x_ref, a_ref, w1_ref, w2_ref, out_ref,
             xr_ref, ar_ref, cs_ref, cr_ref, send_sems, recv_sems):
        my_x = lax.axis_index("x")
        my_y = lax.axis_index("y")
        nbr = (1 - my_x, my_y)

        barrier_sem = pltpu.get_barrier_semaphore()
        pl.semaphore_signal(barrier_sem, inc=1, device_id=nbr,
                            device_id_type=pl.DeviceIdType.MESH)
        pl.semaphore_wait(barrier_sem, 1)

        rdma_x = pltpu.make_async_remote_copy(
            src_ref=x_ref, dst_ref=xr_ref,
            send_sem=send_sems.at[0], recv_sem=recv_sems.at[0],
            device_id=nbr, device_id_type=pl.DeviceIdType.MESH)
        rdma_a = pltpu.make_async_remote_copy(
            src_ref=a_ref, dst_ref=ar_ref,
            send_sem=send_sems.at[1], recv_sem=recv_sems.at[1],
            device_id=nbr, device_id_type=pl.DeviceIdType.MESH)
        rdma_x.start()
        rdma_a.start()

        def contrib(xv, av, k):
            e = 2 * my_x + k
            xe = jnp.where(av == e, xv, 0.0)
            h = jnp.maximum(
                jnp.dot(xe, w1_ref[k], preferred_element_type=jnp.float32),
                0.0)
            return jnp.dot(h, w2_ref[k], preferred_element_type=jnp.float32)

        xv = x_ref[...]
        av = a_ref[...]
        out_ref[...] = contrib(xv, av, 0) + contrib(xv, av, 1)

        rdma_x.wait()
        rdma_a.wait()

        xrv = xr_ref[...]
        arv = ar_ref[...]
        cs_ref[...] = contrib(xrv, arv, 0) + contrib(xrv, arv, 1)

        rdma_c = pltpu.make_async_remote_copy(
            src_ref=cs_ref, dst_ref=cr_ref,
            send_sem=send_sems.at[2], recv_sem=recv_sems.at[2],
            device_id=nbr, device_id_type=pl.DeviceIdType.MESH)
        rdma_c.start()
        rdma_c.wait()

        out_ref[...] = out_ref[...] + cr_ref[...]

    return pl.pallas_call(
        body,
        out_shape=jax.ShapeDtypeStruct((T, D), jnp.float32),
        in_specs=[
            pl.BlockSpec(memory_space=pltpu.VMEM),
            pl.BlockSpec(memory_space=pltpu.VMEM),
            pl.BlockSpec(memory_space=pltpu.VMEM),
            pl.BlockSpec(memory_space=pltpu.VMEM),
        ],
        out_specs=pl.BlockSpec(memory_space=pltpu.VMEM),
        scratch_shapes=[
            pltpu.VMEM((T, D), jnp.float32),
            pltpu.VMEM((T, 1), jnp.int32),
            pltpu.VMEM((T, D), jnp.float32),
            pltpu.VMEM((T, D), jnp.float32),
            pltpu.SemaphoreType.DMA((3,)),
            pltpu.SemaphoreType.DMA((3,)),
        ],
        compiler_params=pltpu.CompilerParams(collective_id=0),
    )(x, assign2, W1, W2)


# device time: 30299 ns/iter; 1.3702x vs baseline; 1.3702x over previous
import jax
import jax.numpy as jnp
from jax import lax
from jax.experimental import pallas as pl
from jax.experimental.pallas import tpu as pltpu

T = 512
D = 512
F = 1024
E_LOC = 2


def kernel(x, assign, W1, W2):
    assign2 = assign.reshape(T, 1)

    def body(x_ref, a_ref, w1_ref, w2_ref, out_ref,
             xb_ref, xr_ref, ar_ref, w1b_ref, w2b_ref, cs_ref, cr_ref,
             send_sems, recv_sems):
        my_x = lax.axis_index("x")
        my_y = lax.axis_index("y")
        nbr = (1 - my_x, my_y)

        barrier_sem = pltpu.get_barrier_semaphore()
        pl.semaphore_signal(barrier_sem, inc=1, device_id=nbr,
                            device_id_type=pl.DeviceIdType.MESH)
        pl.semaphore_wait(barrier_sem, 1)

        xb_ref[...] = x_ref[...].astype(jnp.bfloat16)
        rdma_x = pltpu.make_async_remote_copy(
            src_ref=xb_ref, dst_ref=xr_ref,
            send_sem=send_sems.at[0], recv_sem=recv_sems.at[0],
            device_id=nbr, device_id_type=pl.DeviceIdType.MESH)
        rdma_a = pltpu.make_async_remote_copy(
            src_ref=a_ref, dst_ref=ar_ref,
            send_sem=send_sems.at[1], recv_sem=recv_sems.at[1],
            device_id=nbr, device_id_type=pl.DeviceIdType.MESH)
        rdma_x.start()
        rdma_a.start()

        w1b_ref[...] = w1_ref[...].astype(jnp.bfloat16)
        w2b_ref[...] = w2_ref[...].astype(jnp.bfloat16)

        def contrib(xv, av, k):
            e = 2 * my_x + k
            xe = jnp.where(av == e, xv, jnp.bfloat16(0.0))
            h = jnp.maximum(
                jnp.dot(xe, w1b_ref[k], preferred_element_type=jnp.float32),
                0.0).astype(jnp.bfloat16)
            return jnp.dot(h, w2b_ref[k], preferred_element_type=jnp.float32)

        xv = xb_ref[...]
        av = a_ref[...]
        out_ref[...] = contrib(xv, av, 0) + contrib(xv, av, 1)

        rdma_x.wait()
        rdma_a.wait()

        xrv = xr_ref[...]
        arv = ar_ref[...]
        cs_ref[...] = (contrib(xrv, arv, 0)
                       + contrib(xrv, arv, 1)).astype(jnp.bfloat16)

        rdma_c = pltpu.make_async_remote_copy(
            src_ref=cs_ref, dst_ref=cr_ref,
            send_sem=send_sems.at[2], recv_sem=recv_sems.at[2],
            device_id=nbr, device_id_type=pl.DeviceIdType.MESH)
        rdma_c.start()
        rdma_c.wait()

        out_ref[...] = out_ref[...] + cr_ref[...].astype(jnp.float32)

    return pl.pallas_call(
        body,
        out_shape=jax.ShapeDtypeStruct((T, D), jnp.float32),
        in_specs=[
            pl.BlockSpec(memory_space=pltpu.VMEM),
            pl.BlockSpec(memory_space=pltpu.VMEM),
            pl.BlockSpec(memory_space=pltpu.VMEM),
            pl.BlockSpec(memory_space=pltpu.VMEM),
        ],
        out_specs=pl.BlockSpec(memory_space=pltpu.VMEM),
        scratch_shapes=[
            pltpu.VMEM((T, D), jnp.bfloat16),
            pltpu.VMEM((T, D), jnp.bfloat16),
            pltpu.VMEM((T, 1), jnp.int32),
            pltpu.VMEM((E_LOC, D, F), jnp.bfloat16),
            pltpu.VMEM((E_LOC, F, D), jnp.bfloat16),
            pltpu.VMEM((T, D), jnp.bfloat16),
            pltpu.VMEM((T, D), jnp.bfloat16),
            pltpu.SemaphoreType.DMA((3,)),
            pltpu.SemaphoreType.DMA((3,)),
        ],
        compiler_params=pltpu.CompilerParams(collective_id=0),
    )(x, assign2, W1, W2)


# device time: 18142 ns/iter; 2.2883x vs baseline; 1.6701x over previous
import jax
import jax.numpy as jnp
from jax import lax
from jax.experimental import pallas as pl
from jax.experimental.pallas import tpu as pltpu

T = 512
D = 512
F = 1024
E_LOC = 2
HT = T // 2
NCH = 2
CH = HT // NCH

S_X0 = 0
S_A = NCH
S_CS0 = NCH + 1
S_CD0 = NCH + 1 + NCH
NS = 1 + 3 * NCH


def kernel(x, assign, W1, W2):
    xc = pltpu.with_memory_space_constraint(x, pltpu.MemorySpace.HBM)
    w1c = pltpu.with_memory_space_constraint(W1, pltpu.MemorySpace.HBM)
    w2c = pltpu.with_memory_space_constraint(W2, pltpu.MemorySpace.HBM)
    assign2 = assign.reshape(T, 1)

    def body(xh_ref, a_ref, w1h_ref, w2h_ref, out_ref,
             as_ref, xr_ref, ar_ref, cs_ref, cr_ref,
             xf_ref, xb_ref, w1f_ref, w2f_ref, w1_ref, w2_ref, wsems,
             send_sems, recv_sems):
        my_x = lax.axis_index("x")
        my_y = lax.axis_index("y")
        xn = (1 - my_x, my_y)
        yn = (my_x, 1 - my_y)
        dg = (1 - my_x, 1 - my_y)
        qs = my_y * HT
        qo = (1 - my_y) * HT

        x_dma = pltpu.make_async_copy(xh_ref, xf_ref, wsems.at[2])
        x_dma.start()
        w1_dma = pltpu.make_async_copy(w1h_ref, w1f_ref, wsems.at[0])
        w2_dma = pltpu.make_async_copy(w2h_ref, w2f_ref, wsems.at[1])
        w1_dma.start()
        w2_dma.start()

        barrier_sem = pltpu.get_barrier_semaphore()
        for nbr in (xn, yn, dg):
            pl.semaphore_signal(barrier_sem, inc=1, device_id=nbr,
                                device_id_type=pl.DeviceIdType.MESH)
        pl.semaphore_wait(barrier_sem, 3)

        as_ref[...] = a_ref[pl.ds(qs, HT), :]
        rdma_a = pltpu.make_async_remote_copy(
            src_ref=as_ref, dst_ref=ar_ref,
            send_sem=send_sems.at[S_A], recv_sem=recv_sems.at[S_A],
            device_id=xn, device_id_type=pl.DeviceIdType.MESH)
        rdma_a.start()
        x_dma.wait()
        xb_ref[pl.ds(0, HT), :] = xf_ref[pl.ds(qs, HT), :].astype(
            jnp.bfloat16)
        rdma_x = []
        for c in range(NCH):
            r = pltpu.make_async_remote_copy(
                src_ref=xb_ref.at[pl.ds(c * CH, CH)],
                dst_ref=xr_ref.at[pl.ds(c * CH, CH)],
                send_sem=send_sems.at[S_X0 + c],
                recv_sem=recv_sems.at[S_X0 + c],
                device_id=xn, device_id_type=pl.DeviceIdType.MESH)
            r.start()
            rdma_x.append(r)

        w1_dma.wait()
        w1_ref[...] = w1f_ref[...].astype(jnp.bfloat16)
        w2_dma.wait()
        w2_ref[...] = w2f_ref[...].astype(jnp.bfloat16)

        def contrib(xv, av, k):
            e = 2 * my_x + k
            xe = jnp.where(av == e, xv, jnp.bfloat16(0.0))
            h = jnp.maximum(
                jnp.dot(xe, w1_ref[k], preferred_element_type=jnp.float32),
                0.0).astype(jnp.bfloat16)
            return jnp.dot(h, w2_ref[k], preferred_element_type=jnp.float32)

        av_m = as_ref[...]
        o_mine = contrib(xb_ref[...], av_m, 0) + contrib(xb_ref[...], av_m, 1)

        rdma_a.wait_recv()
        rdma_cs, rdma_cd = [], []
        for c in range(NCH):
            rdma_x[c].wait_recv()
            sl = pl.ds(c * CH, CH)
            arv = ar_ref[sl, :]
            cs_ref[sl, :] = (contrib(xr_ref[sl, :], arv, 0)
                             + contrib(xr_ref[sl, :], arv, 1)
                             ).astype(jnp.bfloat16)
            dst = pl.ds(qs + c * CH, CH)
            rs = pltpu.make_async_remote_copy(
                src_ref=cs_ref.at[sl], dst_ref=cr_ref.at[dst],
                send_sem=send_sems.at[S_CS0 + c],
                recv_sem=recv_sems.at[S_CS0 + c],
                device_id=xn, device_id_type=pl.DeviceIdType.MESH)
            rd = pltpu.make_async_remote_copy(
                src_ref=cs_ref.at[sl], dst_ref=cr_ref.at[dst],
                send_sem=send_sems.at[S_CD0 + c],
                recv_sem=recv_sems.at[S_CD0 + c],
                device_id=dg, device_id_type=pl.DeviceIdType.MESH)
            rs.start()
            rd.start()
            rdma_cs.append(rs)
            rdma_cd.append(rd)

        av_o = a_ref[pl.ds(qo, HT), :]
        xv_o = xf_ref[pl.ds(qo, HT), :].astype(jnp.bfloat16)
        o_other = contrib(xv_o, av_o, 0) + contrib(xv_o, av_o, 1)

        for c in range(NCH):
            rdma_cs[c].wait_recv()
            dst = pl.ds(qs + c * CH, CH)
            out_ref[dst, :] = (o_mine[c * CH:(c + 1) * CH, :]
                               + cr_ref[dst, :].astype(jnp.float32)
                               ).astype(jnp.bfloat16)
        for c in range(NCH):
            rdma_cd[c].wait_recv()
            dst = pl.ds(qo + c * CH, CH)
            out_ref[dst, :] = (o_other[c * CH:(c + 1) * CH, :]
                               + cr_ref[dst, :].astype(jnp.float32)
                               ).astype(jnp.bfloat16)

        for r in rdma_x + rdma_cs + rdma_cd + [rdma_a]:
            r.wait_send()

    return pl.pallas_call(
        body,
        out_shape=jax.ShapeDtypeStruct((T, D), jnp.bfloat16),
        in_specs=[
            pl.BlockSpec(memory_space=pl.ANY),
            pl.BlockSpec(memory_space=pltpu.VMEM),
            pl.BlockSpec(memory_space=pl.ANY),
            pl.BlockSpec(memory_space=pl.ANY),
        ],
        out_specs=pl.BlockSpec(memory_space=pltpu.VMEM),
        scratch_shapes=[
            pltpu.VMEM((HT, 1), jnp.int32),
            pltpu.VMEM((HT, D), jnp.bfloat16),
            pltpu.VMEM((HT, 1), jnp.int32),
            pltpu.VMEM((HT, D), jnp.bfloat16),
            pltpu.VMEM((T, D), jnp.bfloat16),
            pltpu.VMEM((T, D), jnp.float32),
            pltpu.VMEM((HT, D), jnp.bfloat16),
            pltpu.VMEM((E_LOC, D, F), jnp.float32),
            pltpu.VMEM((E_LOC, F, D), jnp.float32),
            pltpu.VMEM((E_LOC, D, F), jnp.bfloat16),
            pltpu.VMEM((E_LOC, F, D), jnp.bfloat16),
            pltpu.SemaphoreType.DMA((3,)),
            pltpu.SemaphoreType.DMA((NS,)),
            pltpu.SemaphoreType.DMA((NS,)),
        ],
        compiler_params=pltpu.CompilerParams(collective_id=0),
    )(xc, assign2, w1c, w2c)
